# baseline (speedup 1.0000x reference)
import functools

import jax
import jax.numpy as jnp
from jax import lax
from jax.experimental import pallas as pl
from jax.experimental.pallas import tpu as pltpu
from jax.experimental.pallas import tpu_sc as plsc

N_NODES = 100000
N_EDGES = 1600000
F_IN = 128
HID = 32
NCLS = 40

LANES = 128
CHUNK = 8
NCH = 100
ROWS_PER_TILE = NCH * CHUNK
ROWS = 16 * ROWS_PER_TILE
HALF = N_NODES // 2
TR = 51200
TPT = TR // 16
SEGS = TPT // 128
DUMMY = HALF

_MESH = dict(core_axis_name="c", subcore_axis_name="s")
_SC_PARAMS = dict(compiler_params=pltpu.CompilerParams(use_tc_tiling_on_sc=False))


def _make_deg_kernel():
    mesh = plsc.VectorSubcoreMesh(**_MESH)

    @functools.partial(
        pl.kernel,
        mesh=mesh,
        out_type=jax.ShapeDtypeStruct((2 * TR,), jnp.float32),
        scratch_types=[
            pltpu.VMEM((4, CHUNK, LANES), jnp.int32),
            pltpu.VMEM((LANES,), jnp.float32),
            pltpu.VMEM((LANES,), jnp.float32),
            pltpu.VMEM_SHARED((TR,), jnp.float32),
            pltpu.SemaphoreType.DMA,
            pltpu.SemaphoreType.DMA,
            pltpu.SemaphoreType.DMA,
            pltpu.SemaphoreType.DMA,
        ],
        **_SC_PARAMS,
    )
    def deg_kernel(loc_hbm, out_hbm, loc_v, ones_v, zb_v, table,
                   isem0, isem1, ssem0, ssem1):
        c = lax.axis_index("c")
        s = lax.axis_index("s")
        isems = (isem0, isem1)
        ssems = (ssem0, ssem1)
        row0 = s * ROWS_PER_TILE

        pltpu.async_copy(loc_hbm.at[c, pl.ds(row0, CHUNK)], loc_v.at[0], isems[0])
        pltpu.async_copy(loc_hbm.at[c, pl.ds(row0 + CHUNK, CHUNK)],
                         loc_v.at[1], isems[1])

        for q in range(LANES // 16):
            zb_v[pl.ds(q * 16, 16)] = jnp.zeros((16,), jnp.float32)
            ones_v[pl.ds(q * 16, 16)] = jnp.ones((16,), jnp.float32)
        tb = s * TPT
        for k in range(SEGS):
            pltpu.sync_copy(zb_v, table.at[pl.ds(tb + k * 128, 128)])
        plsc.subcore_barrier()

        def body(tt, carry):
            for k in range(4):
                g = k % 2
                b2 = (k + 2) % 4
                b3 = (k + 3) % 4
                t = 4 * tt + k
                rt = row0 + t * CHUNK
                pltpu.make_async_copy(loc_hbm.at[c, pl.ds(rt, CHUNK)],
                                      loc_v.at[k], isems[g]).wait()

                def drain_prev():
                    for i in range(CHUNK):
                        pltpu.make_async_copy(
                            ones_v, table.at[loc_v.at[b3, i]], ssems[1 - g]
                        ).wait()
                if k == 0:
                    @pl.when(tt > 0)
                    def _():
                        drain_prev()
                else:
                    drain_prev()

                for i in range(CHUNK):
                    pltpu.async_copy(ones_v, table.at[loc_v.at[k, i]],
                                     ssems[g], add=True)
                r2 = row0 + jnp.minimum(t + 2, NCH - 1) * CHUNK
                pltpu.async_copy(loc_hbm.at[c, pl.ds(r2, CHUNK)],
                                 loc_v.at[b2], isems[g])
            return carry

        lax.fori_loop(0, NCH // 4, body, 0)

        rl = row0 + (NCH - 1) * CHUNK
        for i in range(CHUNK):
            pltpu.make_async_copy(ones_v, table.at[loc_v.at[3, i]],
                                  ssems[1]).wait()
        pltpu.make_async_copy(loc_hbm.at[c, pl.ds(rl, CHUNK)],
                              loc_v.at[0], isems[0]).wait()
        pltpu.make_async_copy(loc_hbm.at[c, pl.ds(rl, CHUNK)],
                              loc_v.at[1], isems[1]).wait()

        plsc.subcore_barrier()
        for k in range(SEGS):
            pltpu.sync_copy(table.at[pl.ds(tb + k * 128, 128)],
                            out_hbm.at[pl.ds(c * TR + tb + k * 128, 128)])

    return deg_kernel


GR = 8
SW = 2
NG = ROWS_PER_TILE // GR


def _make_agg_kernel(D):
    mesh = plsc.VectorSubcoreMesh(**_MESH)

    @functools.partial(
        pl.kernel,
        mesh=mesh,
        out_type=jax.ShapeDtypeStruct((2, TR, D), jnp.float32),
        scratch_types=[
            pltpu.VMEM((4, GR, LANES), jnp.int32),
            pltpu.VMEM((4, GR, LANES), jnp.int32),
            pltpu.VMEM((2, SW, LANES, D), jnp.float32),
            pltpu.VMEM_SHARED((TR, D), jnp.float32),
            pltpu.SemaphoreType.DMA,
            pltpu.SemaphoreType.DMA,
            pltpu.SemaphoreType.DMA,
            pltpu.SemaphoreType.DMA,
            pltpu.SemaphoreType.DMA,
            pltpu.SemaphoreType.DMA,
        ],
        **_SC_PARAMS,
    )
    def agg_kernel(hs_hbm, src_hbm, loc_hbm, zeros_hbm, out_hbm,
                   src_v, loc_v, gbufs, table,
                   gsem0, gsem1, ssem0, ssem1, isem0, isem1):
        c = lax.axis_index("c")
        s = lax.axis_index("s")
        gsems = (gsem0, gsem1)
        ssems = (ssem0, ssem1)
        isems = (isem0, isem1)
        row0 = s * ROWS_PER_TILE
        tb = s * TPT

        def idx_issue(gexpr, bank, sem):
            r = row0 + jnp.minimum(gexpr, NG - 1) * GR
            pltpu.async_copy(src_hbm.at[pl.ds(r, GR)], src_v.at[bank], sem)
            pltpu.async_copy(loc_hbm.at[c, pl.ds(r, GR)], loc_v.at[bank], sem)

        def idx_wait(gexpr, bank, sem):
            r = row0 + jnp.minimum(gexpr, NG - 1) * GR
            pltpu.make_async_copy(src_hbm.at[pl.ds(r, GR)],
                                  src_v.at[bank], sem).wait()
            pltpu.make_async_copy(loc_hbm.at[c, pl.ds(r, GR)],
                                  loc_v.at[bank], sem).wait()

        idx_issue(0, 0, isems[0])
        idx_issue(1, 1, isems[1])
        for k in range(SEGS):
            pltpu.sync_copy(zeros_hbm, table.at[pl.ds(tb + k * 128, 128)])
        plsc.subcore_barrier()

        idx_wait(0, 0, isems[0])
        for i in range(SW):
            pltpu.async_copy(hs_hbm.at[src_v.at[0, i]], gbufs.at[0, i],
                             gsems[0])

        def body(tt, carry):
            for gg in range(4):
                G = 4 * tt + gg
                idx_wait(G + 1, (gg + 1) % 4, isems[(gg + 1) % 2])
                idx_issue(G + 2, (gg + 2) % 4, isems[gg % 2])
                for q in range(4):
                    su = q % 2
                    for i in range(SW):
                        pltpu.make_async_copy(
                            hs_hbm.at[src_v.at[gg, SW * q + i]],
                            gbufs.at[su, i], gsems[su]).wait()
                    for i in range(SW):
                        pltpu.async_copy(
                            gbufs.at[su, i],
                            table.at[loc_v.at[gg, SW * q + i]],
                            ssems[su], add=True)
                    pb, pr = (gg, SW * (q - 1)) if q else ((gg + 3) % 4,
                                                           SW * 3)

                    def drain_prev(pb=pb, pr=pr, su=su):
                        for i in range(SW):
                            pltpu.make_async_copy(
                                gbufs.at[1 - su, i],
                                table.at[loc_v.at[pb, pr + i]],
                                ssems[1 - su]).wait()
                    if gg == 0 and q == 0:
                        @pl.when(tt > 0)
                        def _():
                            drain_prev()
                    else:
                        drain_prev()
                    nb, nr = (gg, SW * (q + 1)) if q < 3 else ((gg + 1) % 4, 0)
                    for i in range(SW):
                        pltpu.async_copy(hs_hbm.at[src_v.at[nb, nr + i]],
                                         gbufs.at[1 - su, i], gsems[1 - su])
            return carry

        lax.fori_loop(0, NG // 4, body, 0)

        for i in range(SW):
            pltpu.make_async_copy(hs_hbm.at[src_v.at[0, i]],
                                  gbufs.at[0, i], gsems[0]).wait()
        for i in range(SW):
            pltpu.make_async_copy(gbufs.at[1, i],
                                  table.at[loc_v.at[3, SW * 3 + i]],
                                  ssems[1]).wait()
        idx_wait(NG - 1, 1, isems[1])

        plsc.subcore_barrier()
        for k in range(SEGS):
            pltpu.sync_copy(table.at[pl.ds(tb + k * 128, 128)],
                            out_hbm.at[c, pl.ds(tb + k * 128, 128)])

    return agg_kernel


_deg_call = _make_deg_kernel()
_agg32_call = _make_agg_kernel(HID)


BLK = 2000
RB = ROWS // 16


def _loc_body(dst_ref, out_ref):
    i = pl.program_id(0)
    l = dst_ref[...] - i * HALF
    ok = (l >= 0) & (l < HALF)
    out_ref[...] = jnp.where(ok, l, DUMMY)[None]


def _loc_call(dstp):
    return pl.pallas_call(
        _loc_body,
        grid=(2, 16),
        in_specs=[pl.BlockSpec((RB, LANES), lambda i, j: (j, 0))],
        out_specs=pl.BlockSpec((1, RB, LANES), lambda i, j: (i, j, 0)),
        out_shape=jax.ShapeDtypeStruct((2, ROWS, LANES), jnp.int32),
    )(dstp)


def _pre_body(x_ref, w_ref, deg_ref, hs_ref, s_ref):
    s = lax.rsqrt(deg_ref[0] + 1.0)
    h = jnp.dot(x_ref[...], w_ref[...], preferred_element_type=jnp.float32)
    hs_ref[...] = h * s
    s_ref[...] = s


def _pre_call(x, W1, deg3):
    return pl.pallas_call(
        _pre_body,
        grid=(2, HALF // BLK),
        in_specs=[
            pl.BlockSpec((BLK, F_IN), lambda i, j: (i * (HALF // BLK) + j, 0)),
            pl.BlockSpec((F_IN, HID), lambda i, j: (0, 0)),
            pl.BlockSpec((1, BLK, 1), lambda i, j: (i, j, 0)),
        ],
        out_specs=[
            pl.BlockSpec((BLK, HID), lambda i, j: (i * (HALF // BLK) + j, 0)),
            pl.BlockSpec((BLK, 1), lambda i, j: (i * (HALF // BLK) + j, 0)),
        ],
        out_shape=[
            jax.ShapeDtypeStruct((N_NODES, HID), jnp.float32),
            jax.ShapeDtypeStruct((N_NODES, 1), jnp.float32),
        ],
    )(x, W1, deg3)


def _mid_body(agg_ref, hs_ref, s_ref, b_ref, out_ref):
    s = s_ref[...]
    z = jnp.maximum(s * (agg_ref[0] + hs_ref[...]) + b_ref[...], 0.0)
    out_ref[...] = z * s


def _mid_call(agg1, hs1, sinv, b1r):
    return pl.pallas_call(
        _mid_body,
        grid=(2, HALF // BLK),
        in_specs=[
            pl.BlockSpec((1, BLK, HID), lambda i, j: (i, j, 0)),
            pl.BlockSpec((BLK, HID), lambda i, j: (i * (HALF // BLK) + j, 0)),
            pl.BlockSpec((BLK, 1), lambda i, j: (i * (HALF // BLK) + j, 0)),
            pl.BlockSpec((1, HID), lambda i, j: (0, 0)),
        ],
        out_specs=pl.BlockSpec((BLK, HID),
                               lambda i, j: (i * (HALF // BLK) + j, 0)),
        out_shape=jax.ShapeDtypeStruct((N_NODES, HID), jnp.float32),
    )(agg1, hs1, sinv, b1r)


def _post_body(agg_ref, zs_ref, s_ref, w_ref, b_ref, out_ref):
    s = s_ref[...]
    u = s * (agg_ref[0] + zs_ref[...])
    h2 = jnp.dot(u, w_ref[...], preferred_element_type=jnp.float32)
    z = jnp.maximum(h2 + b_ref[...], 0.0)
    m = jnp.max(z, axis=1, keepdims=True)
    lse = jnp.log(jnp.sum(jnp.exp(z - m), axis=1, keepdims=True)) + m
    out_ref[...] = z - lse


def _post_call(agg2, zs, sinv, W2, b2r):
    return pl.pallas_call(
        _post_body,
        grid=(2, HALF // BLK),
        in_specs=[
            pl.BlockSpec((1, BLK, HID), lambda i, j: (i, j, 0)),
            pl.BlockSpec((BLK, HID), lambda i, j: (i * (HALF // BLK) + j, 0)),
            pl.BlockSpec((BLK, 1), lambda i, j: (i * (HALF // BLK) + j, 0)),
            pl.BlockSpec((HID, NCLS), lambda i, j: (0, 0)),
            pl.BlockSpec((1, NCLS), lambda i, j: (0, 0)),
        ],
        out_specs=pl.BlockSpec((BLK, NCLS),
                               lambda i, j: (i * (HALF // BLK) + j, 0)),
        out_shape=jax.ShapeDtypeStruct((N_NODES, NCLS), jnp.float32),
    )(agg2, zs, sinv, W2, b2r)


def kernel(x, edge_index, W1, b1, W2, b2):
    pad = ROWS * LANES - N_EDGES
    srcp = jnp.concatenate(
        [edge_index[0], jnp.zeros((pad,), jnp.int32)]).reshape(ROWS, LANES)
    dstp = jnp.concatenate(
        [edge_index[1], jnp.full((pad,), N_NODES, jnp.int32)]).reshape(ROWS, LANES)

    locp = _loc_call(dstp)
    deg2 = _deg_call(locp).reshape(2, TR)
    deg3 = deg2[:, :HALF].reshape(2, HALF, 1)

    hs1, sinv = _pre_call(x, W1, deg3)
    zeros32 = jnp.zeros((LANES, HID), jnp.float32)
    agg1 = _agg32_call(hs1, srcp, locp, zeros32)

    zs = _mid_call(agg1, hs1, sinv, b1.reshape(1, HID))
    agg2 = _agg32_call(zs, srcp, locp, zeros32)

    return _post_call(agg2, zs, sinv, W2, b2.reshape(1, NCLS))


# speedup vs baseline: 1.5801x; 1.5801x over previous
import functools

import jax
import jax.numpy as jnp
from jax import lax
from jax.experimental import pallas as pl
from jax.experimental.pallas import tpu as pltpu
from jax.experimental.pallas import tpu_sc as plsc

N_NODES = 100000
N_EDGES = 1600000
F_IN = 128
HID = 32
NCLS = 40

LANES = 128
CHUNK = 8
NCH = 100
ROWS_PER_TILE = NCH * CHUNK
ROWS = 16 * ROWS_PER_TILE
HALF = N_NODES // 2
TR = 51200
TPT = TR // 16
SEGS = TPT // 128
DUMMY = HALF

_MESH = dict(core_axis_name="c", subcore_axis_name="s")
_SC_PARAMS = dict(compiler_params=pltpu.CompilerParams(use_tc_tiling_on_sc=False))


def _make_deg_kernel():
    mesh = plsc.VectorSubcoreMesh(**_MESH)

    @functools.partial(
        pl.kernel,
        mesh=mesh,
        out_type=jax.ShapeDtypeStruct((2 * TR,), jnp.float32),
        scratch_types=[
            pltpu.VMEM((4, CHUNK, LANES), jnp.int32),
            pltpu.VMEM((LANES,), jnp.float32),
            pltpu.VMEM((LANES,), jnp.float32),
            pltpu.VMEM_SHARED((TR,), jnp.float32),
            pltpu.SemaphoreType.DMA,
            pltpu.SemaphoreType.DMA,
            pltpu.SemaphoreType.DMA,
            pltpu.SemaphoreType.DMA,
        ],
        **_SC_PARAMS,
    )
    def deg_kernel(loc_hbm, out_hbm, loc_v, ones_v, zb_v, table,
                   isem0, isem1, ssem0, ssem1):
        c = lax.axis_index("c")
        s = lax.axis_index("s")
        isems = (isem0, isem1)
        ssems = (ssem0, ssem1)
        row0 = s * ROWS_PER_TILE

        pltpu.async_copy(loc_hbm.at[c, pl.ds(row0, CHUNK)], loc_v.at[0], isems[0])
        pltpu.async_copy(loc_hbm.at[c, pl.ds(row0 + CHUNK, CHUNK)],
                         loc_v.at[1], isems[1])

        for q in range(LANES // 16):
            zb_v[pl.ds(q * 16, 16)] = jnp.zeros((16,), jnp.float32)
            ones_v[pl.ds(q * 16, 16)] = jnp.ones((16,), jnp.float32)
        tb = s * TPT
        for k in range(SEGS):
            pltpu.sync_copy(zb_v, table.at[pl.ds(tb + k * 128, 128)])
        plsc.subcore_barrier()

        def body(tt, carry):
            for k in range(4):
                g = k % 2
                b2 = (k + 2) % 4
                b3 = (k + 3) % 4
                t = 4 * tt + k
                rt = row0 + t * CHUNK
                pltpu.make_async_copy(loc_hbm.at[c, pl.ds(rt, CHUNK)],
                                      loc_v.at[k], isems[g]).wait()

                def drain_prev():
                    for i in range(CHUNK):
                        pltpu.make_async_copy(
                            ones_v, table.at[loc_v.at[b3, i]], ssems[1 - g]
                        ).wait()
                if k == 0:
                    @pl.when(tt > 0)
                    def _():
                        drain_prev()
                else:
                    drain_prev()

                for i in range(CHUNK):
                    pltpu.async_copy(ones_v, table.at[loc_v.at[k, i]],
                                     ssems[g], add=True)
                r2 = row0 + jnp.minimum(t + 2, NCH - 1) * CHUNK
                pltpu.async_copy(loc_hbm.at[c, pl.ds(r2, CHUNK)],
                                 loc_v.at[b2], isems[g])
            return carry

        lax.fori_loop(0, NCH // 4, body, 0)

        rl = row0 + (NCH - 1) * CHUNK
        for i in range(CHUNK):
            pltpu.make_async_copy(ones_v, table.at[loc_v.at[3, i]],
                                  ssems[1]).wait()
        pltpu.make_async_copy(loc_hbm.at[c, pl.ds(rl, CHUNK)],
                              loc_v.at[0], isems[0]).wait()
        pltpu.make_async_copy(loc_hbm.at[c, pl.ds(rl, CHUNK)],
                              loc_v.at[1], isems[1]).wait()

        plsc.subcore_barrier()
        for k in range(SEGS):
            pltpu.sync_copy(table.at[pl.ds(tb + k * 128, 128)],
                            out_hbm.at[pl.ds(c * TR + tb + k * 128, 128)])

    return deg_kernel


GR = 8
SW = 2
NG = ROWS_PER_TILE // GR


def _make_agg_kernel(D):
    mesh = plsc.VectorSubcoreMesh(**_MESH)

    @functools.partial(
        pl.kernel,
        mesh=mesh,
        out_type=jax.ShapeDtypeStruct((2, TR, D), jnp.float32),
        scratch_types=[
            pltpu.VMEM((4, GR, LANES), jnp.int32),
            pltpu.VMEM((4, GR, LANES), jnp.int32),
            pltpu.VMEM((2, SW, LANES, D), jnp.float32),
            pltpu.VMEM_SHARED((TR, D), jnp.float32),
            pltpu.SemaphoreType.DMA,
            pltpu.SemaphoreType.DMA,
            pltpu.SemaphoreType.DMA,
            pltpu.SemaphoreType.DMA,
            pltpu.SemaphoreType.DMA,
            pltpu.SemaphoreType.DMA,
        ],
        **_SC_PARAMS,
    )
    def agg_kernel(hs_hbm, src_hbm, loc_hbm, zeros_hbm, out_hbm,
                   src_v, loc_v, gbufs, table,
                   gsem0, gsem1, ssem0, ssem1, isem0, isem1):
        c = lax.axis_index("c")
        s = lax.axis_index("s")
        gsems = (gsem0, gsem1)
        ssems = (ssem0, ssem1)
        isems = (isem0, isem1)
        row0 = s * ROWS_PER_TILE
        tb = s * TPT

        def idx_issue(gexpr, bank, sem):
            r = row0 + jnp.minimum(gexpr, NG - 1) * GR
            pltpu.async_copy(src_hbm.at[pl.ds(r, GR)], src_v.at[bank], sem)
            pltpu.async_copy(loc_hbm.at[c, pl.ds(r, GR)], loc_v.at[bank], sem)

        def idx_wait(gexpr, bank, sem):
            r = row0 + jnp.minimum(gexpr, NG - 1) * GR
            pltpu.make_async_copy(src_hbm.at[pl.ds(r, GR)],
                                  src_v.at[bank], sem).wait()
            pltpu.make_async_copy(loc_hbm.at[c, pl.ds(r, GR)],
                                  loc_v.at[bank], sem).wait()

        idx_issue(0, 0, isems[0])
        idx_issue(1, 1, isems[1])
        for k in range(SEGS):
            pltpu.sync_copy(zeros_hbm, table.at[pl.ds(tb + k * 128, 128)])
        plsc.subcore_barrier()

        idx_wait(0, 0, isems[0])
        for i in range(SW):
            pltpu.async_copy(hs_hbm.at[src_v.at[0, i]], gbufs.at[0, i],
                             gsems[0])

        def body(tt, carry):
            for gg in range(4):
                G = 4 * tt + gg
                idx_wait(G + 1, (gg + 1) % 4, isems[(gg + 1) % 2])
                idx_issue(G + 2, (gg + 2) % 4, isems[gg % 2])
                for q in range(4):
                    su = q % 2
                    for i in range(SW):
                        pltpu.make_async_copy(
                            hs_hbm.at[src_v.at[gg, SW * q + i]],
                            gbufs.at[su, i], gsems[su]).wait()
                    for i in range(SW):
                        pltpu.async_copy(
                            gbufs.at[su, i],
                            table.at[loc_v.at[gg, SW * q + i]],
                            ssems[su], add=True)
                    pb, pr = (gg, SW * (q - 1)) if q else ((gg + 3) % 4,
                                                           SW * 3)

                    def drain_prev(pb=pb, pr=pr, su=su):
                        for i in range(SW):
                            pltpu.make_async_copy(
                                gbufs.at[1 - su, i],
                                table.at[loc_v.at[pb, pr + i]],
                                ssems[1 - su]).wait()
                    if gg == 0 and q == 0:
                        @pl.when(tt > 0)
                        def _():
                            drain_prev()
                    else:
                        drain_prev()
                    nb, nr = (gg, SW * (q + 1)) if q < 3 else ((gg + 1) % 4, 0)
                    for i in range(SW):
                        pltpu.async_copy(hs_hbm.at[src_v.at[nb, nr + i]],
                                         gbufs.at[1 - su, i], gsems[1 - su])
            return carry

        lax.fori_loop(0, NG // 4, body, 0)

        for i in range(SW):
            pltpu.make_async_copy(hs_hbm.at[src_v.at[0, i]],
                                  gbufs.at[0, i], gsems[0]).wait()
        for i in range(SW):
            pltpu.make_async_copy(gbufs.at[1, i],
                                  table.at[loc_v.at[3, SW * 3 + i]],
                                  ssems[1]).wait()
        idx_wait(NG - 1, 1, isems[1])

        plsc.subcore_barrier()
        for k in range(SEGS):
            pltpu.sync_copy(table.at[pl.ds(tb + k * 128, 128)],
                            out_hbm.at[c, pl.ds(tb + k * 128, 128)])

    return agg_kernel


_deg_call = _make_deg_kernel()
_agg32_call = _make_agg_kernel(HID)


BLK = 2000
RB = ROWS // 16


def _loc_body(dst_ref, out_ref):
    i = pl.program_id(0)
    l = dst_ref[...] - i * HALF
    ok = (l >= 0) & (l < HALF)
    r = lax.broadcasted_iota(jnp.int32, l.shape, 0)
    q = lax.broadcasted_iota(jnp.int32, l.shape, 1)
    trash = DUMMY + ((r * LANES + q) & 1023)
    out_ref[...] = jnp.where(ok, l, trash)[None]


def _loc_call(dstp):
    return pl.pallas_call(
        _loc_body,
        grid=(2, 16),
        in_specs=[pl.BlockSpec((RB, LANES), lambda i, j: (j, 0))],
        out_specs=pl.BlockSpec((1, RB, LANES), lambda i, j: (i, j, 0)),
        out_shape=jax.ShapeDtypeStruct((2, ROWS, LANES), jnp.int32),
    )(dstp)


def _pre_body(x_ref, w_ref, deg_ref, hs_ref, s_ref):
    s = lax.rsqrt(deg_ref[0] + 1.0)
    h = jnp.dot(x_ref[...], w_ref[...], preferred_element_type=jnp.float32)
    hs_ref[...] = h * s
    s_ref[...] = s


def _pre_call(x, W1, deg3):
    return pl.pallas_call(
        _pre_body,
        grid=(2, HALF // BLK),
        in_specs=[
            pl.BlockSpec((BLK, F_IN), lambda i, j: (i * (HALF // BLK) + j, 0)),
            pl.BlockSpec((F_IN, HID), lambda i, j: (0, 0)),
            pl.BlockSpec((1, BLK, 1), lambda i, j: (i, j, 0)),
        ],
        out_specs=[
            pl.BlockSpec((BLK, HID), lambda i, j: (i * (HALF // BLK) + j, 0)),
            pl.BlockSpec((BLK, 1), lambda i, j: (i * (HALF // BLK) + j, 0)),
        ],
        out_shape=[
            jax.ShapeDtypeStruct((N_NODES, HID), jnp.float32),
            jax.ShapeDtypeStruct((N_NODES, 1), jnp.float32),
        ],
    )(x, W1, deg3)


def _mid_body(agg_ref, hs_ref, s_ref, b_ref, out_ref):
    s = s_ref[...]
    z = jnp.maximum(s * (agg_ref[0] + hs_ref[...]) + b_ref[...], 0.0)
    out_ref[...] = z * s


def _mid_call(agg1, hs1, sinv, b1r):
    return pl.pallas_call(
        _mid_body,
        grid=(2, HALF // BLK),
        in_specs=[
            pl.BlockSpec((1, BLK, HID), lambda i, j: (i, j, 0)),
            pl.BlockSpec((BLK, HID), lambda i, j: (i * (HALF // BLK) + j, 0)),
            pl.BlockSpec((BLK, 1), lambda i, j: (i * (HALF // BLK) + j, 0)),
            pl.BlockSpec((1, HID), lambda i, j: (0, 0)),
        ],
        out_specs=pl.BlockSpec((BLK, HID),
                               lambda i, j: (i * (HALF // BLK) + j, 0)),
        out_shape=jax.ShapeDtypeStruct((N_NODES, HID), jnp.float32),
    )(agg1, hs1, sinv, b1r)


def _post_body(agg_ref, zs_ref, s_ref, w_ref, b_ref, out_ref):
    s = s_ref[...]
    u = s * (agg_ref[0] + zs_ref[...])
    h2 = jnp.dot(u, w_ref[...], preferred_element_type=jnp.float32)
    z = jnp.maximum(h2 + b_ref[...], 0.0)
    m = jnp.max(z, axis=1, keepdims=True)
    lse = jnp.log(jnp.sum(jnp.exp(z - m), axis=1, keepdims=True)) + m
    out_ref[...] = z - lse


def _post_call(agg2, zs, sinv, W2, b2r):
    return pl.pallas_call(
        _post_body,
        grid=(2, HALF // BLK),
        in_specs=[
            pl.BlockSpec((1, BLK, HID), lambda i, j: (i, j, 0)),
            pl.BlockSpec((BLK, HID), lambda i, j: (i * (HALF // BLK) + j, 0)),
            pl.BlockSpec((BLK, 1), lambda i, j: (i * (HALF // BLK) + j, 0)),
            pl.BlockSpec((HID, NCLS), lambda i, j: (0, 0)),
            pl.BlockSpec((1, NCLS), lambda i, j: (0, 0)),
        ],
        out_specs=pl.BlockSpec((BLK, NCLS),
                               lambda i, j: (i * (HALF // BLK) + j, 0)),
        out_shape=jax.ShapeDtypeStruct((N_NODES, NCLS), jnp.float32),
    )(agg2, zs, sinv, W2, b2r)


def kernel(x, edge_index, W1, b1, W2, b2):
    pad = ROWS * LANES - N_EDGES
    srcp = jnp.concatenate(
        [edge_index[0], jnp.zeros((pad,), jnp.int32)]).reshape(ROWS, LANES)
    dstp = jnp.concatenate(
        [edge_index[1], jnp.full((pad,), N_NODES, jnp.int32)]).reshape(ROWS, LANES)

    locp = _loc_call(dstp)
    deg2 = _deg_call(locp).reshape(2, TR)
    deg3 = deg2[:, :HALF].reshape(2, HALF, 1)

    hs1, sinv = _pre_call(x, W1, deg3)
    zeros32 = jnp.zeros((LANES, HID), jnp.float32)
    agg1 = _agg32_call(hs1, srcp, locp, zeros32)

    zs = _mid_call(agg1, hs1, sinv, b1.reshape(1, HID))
    agg2 = _agg32_call(zs, srcp, locp, zeros32)

    return _post_call(agg2, zs, sinv, W2, b2.reshape(1, NCLS))


# speedup vs baseline: 2.1754x; 1.3767x over previous
import functools

import jax
import jax.numpy as jnp
from jax import lax
from jax.experimental import pallas as pl
from jax.experimental.pallas import tpu as pltpu
from jax.experimental.pallas import tpu_sc as plsc

N_NODES = 100000
N_EDGES = 1600000
F_IN = 128
HID = 32
NCLS = 40

LANES = 128
CHUNK = 8
NCH = 100
ROWS_PER_TILE = NCH * CHUNK
ROWS = 16 * ROWS_PER_TILE
HALF = N_NODES // 2
TR = 51200
TPT = TR // 16
SEGS = TPT // 128
DUMMY = HALF
ACH = 2
ANCH = ROWS_PER_TILE // ACH
PADZ = 128

_MESH = dict(core_axis_name="c", subcore_axis_name="s")
_SC_PARAMS = dict(compiler_params=pltpu.CompilerParams(use_tc_tiling_on_sc=False))


def _make_deg_kernel():
    mesh = plsc.VectorSubcoreMesh(**_MESH)

    @functools.partial(
        pl.kernel,
        mesh=mesh,
        out_type=jax.ShapeDtypeStruct((2 * TR,), jnp.float32),
        scratch_types=[
            pltpu.VMEM((4, CHUNK, LANES), jnp.int32),
            pltpu.VMEM((LANES,), jnp.float32),
            pltpu.VMEM((LANES,), jnp.float32),
            pltpu.VMEM_SHARED((TR,), jnp.float32),
            pltpu.SemaphoreType.DMA,
            pltpu.SemaphoreType.DMA,
            pltpu.SemaphoreType.DMA,
            pltpu.SemaphoreType.DMA,
        ],
        **_SC_PARAMS,
    )
    def deg_kernel(loc_hbm, out_hbm, loc_v, ones_v, zb_v, table,
                   isem0, isem1, ssem0, ssem1):
        c = lax.axis_index("c")
        s = lax.axis_index("s")
        isems = (isem0, isem1)
        ssems = (ssem0, ssem1)
        row0 = s * ROWS_PER_TILE

        pltpu.async_copy(loc_hbm.at[c, pl.ds(row0, CHUNK)], loc_v.at[0], isems[0])
        pltpu.async_copy(loc_hbm.at[c, pl.ds(row0 + CHUNK, CHUNK)],
                         loc_v.at[1], isems[1])

        for q in range(LANES // 16):
            zb_v[pl.ds(q * 16, 16)] = jnp.zeros((16,), jnp.float32)
            ones_v[pl.ds(q * 16, 16)] = jnp.ones((16,), jnp.float32)
        tb = s * TPT
        for k in range(SEGS):
            pltpu.sync_copy(zb_v, table.at[pl.ds(tb + k * 128, 128)])
        plsc.subcore_barrier()

        def body(tt, carry):
            for k in range(4):
                g = k % 2
                b2 = (k + 2) % 4
                b3 = (k + 3) % 4
                t = 4 * tt + k
                rt = row0 + t * CHUNK
                pltpu.make_async_copy(loc_hbm.at[c, pl.ds(rt, CHUNK)],
                                      loc_v.at[k], isems[g]).wait()

                def drain_prev():
                    for i in range(CHUNK):
                        pltpu.make_async_copy(
                            ones_v, table.at[loc_v.at[b3, i]], ssems[1 - g]
                        ).wait()
                if k == 0:
                    @pl.when(tt > 0)
                    def _():
                        drain_prev()
                else:
                    drain_prev()

                for i in range(CHUNK):
                    pltpu.async_copy(ones_v, table.at[loc_v.at[k, i]],
                                     ssems[g], add=True)
                r2 = row0 + jnp.minimum(t + 2, NCH - 1) * CHUNK
                pltpu.async_copy(loc_hbm.at[c, pl.ds(r2, CHUNK)],
                                 loc_v.at[b2], isems[g])
            return carry

        lax.fori_loop(0, NCH // 4, body, 0)

        rl = row0 + (NCH - 1) * CHUNK
        for i in range(CHUNK):
            pltpu.make_async_copy(ones_v, table.at[loc_v.at[3, i]],
                                  ssems[1]).wait()
        pltpu.make_async_copy(loc_hbm.at[c, pl.ds(rl, CHUNK)],
                              loc_v.at[0], isems[0]).wait()
        pltpu.make_async_copy(loc_hbm.at[c, pl.ds(rl, CHUNK)],
                              loc_v.at[1], isems[1]).wait()

        plsc.subcore_barrier()
        for k in range(SEGS):
            pltpu.sync_copy(table.at[pl.ds(tb + k * 128, 128)],
                            out_hbm.at[pl.ds(c * TR + tb + k * 128, 128)])

    return deg_kernel


def _make_agg_kernel(D):
    mesh = plsc.VectorSubcoreMesh(**_MESH)

    @functools.partial(
        pl.kernel,
        mesh=mesh,
        out_type=jax.ShapeDtypeStruct((2, TR, D), jnp.float32),
        scratch_types=[
            pltpu.VMEM((4, ACH, LANES), jnp.int32),
            pltpu.VMEM((4, ACH, LANES), jnp.int32),
            pltpu.VMEM((2, ACH, LANES, D), jnp.float32),
            pltpu.VMEM_SHARED((TR, D), jnp.float32),
            pltpu.SemaphoreType.DMA,
            pltpu.SemaphoreType.DMA,
            pltpu.SemaphoreType.DMA,
            pltpu.SemaphoreType.DMA,
            pltpu.SemaphoreType.DMA,
            pltpu.SemaphoreType.DMA,
        ],
        **_SC_PARAMS,
    )
    def agg_kernel(hs_hbm, src_hbm, loc_hbm, zeros_hbm, out_hbm,
                   src_v, loc_v, gbufs, table,
                   gsem0, gsem1, ssem0, ssem1, isem0, isem1):
        c = lax.axis_index("c")
        s = lax.axis_index("s")
        gsems = (gsem0, gsem1)
        ssems = (ssem0, ssem1)
        isems = (isem0, isem1)
        row0 = s * ROWS_PER_TILE
        tb = s * TPT

        def idx_issue(texpr, bank, sem):
            r = row0 + jnp.minimum(texpr, ANCH - 1) * ACH
            pltpu.async_copy(src_hbm.at[c, pl.ds(r, ACH)], src_v.at[bank], sem)
            pltpu.async_copy(loc_hbm.at[c, pl.ds(r, ACH)], loc_v.at[bank], sem)

        def idx_wait(texpr, bank, sem):
            r = row0 + jnp.minimum(texpr, ANCH - 1) * ACH
            pltpu.make_async_copy(src_hbm.at[c, pl.ds(r, ACH)],
                                  src_v.at[bank], sem).wait()
            pltpu.make_async_copy(loc_hbm.at[c, pl.ds(r, ACH)],
                                  loc_v.at[bank], sem).wait()

        idx_issue(0, 0, isems[0])
        idx_issue(1, 1, isems[1])
        for k in range(SEGS):
            pltpu.sync_copy(zeros_hbm, table.at[pl.ds(tb + k * 128, 128)])
        plsc.subcore_barrier()

        idx_wait(0, 0, isems[0])
        idx_issue(2, 2, isems[0])
        for i in range(ACH):
            pltpu.async_copy(hs_hbm.at[src_v.at[0, i]], gbufs.at[0, i],
                             gsems[0])

        def body(tt, carry):
            for k in range(4):
                t = 4 * tt + k
                g = k % 2
                h = (k + 1) % 2
                idx_wait(t + 1, (k + 1) % 4, isems[h])
                for i in range(ACH):
                    pltpu.make_async_copy(hs_hbm.at[src_v.at[k, i]],
                                          gbufs.at[g, i], gsems[g]).wait()
                for i in range(ACH):
                    pltpu.async_copy(gbufs.at[g, i], table.at[loc_v.at[k, i]],
                                     ssems[g], add=True)

                def drain_prev():
                    for i in range(ACH):
                        pltpu.make_async_copy(
                            gbufs.at[h, i],
                            table.at[loc_v.at[(k + 3) % 4, i]],
                            ssems[h]).wait()
                if k == 0:
                    @pl.when(tt > 0)
                    def _():
                        drain_prev()
                else:
                    drain_prev()

                idx_issue(t + 3, (k + 3) % 4, isems[h])
                for i in range(ACH):
                    pltpu.async_copy(hs_hbm.at[src_v.at[(k + 1) % 4, i]],
                                     gbufs.at[h, i], gsems[h])
            return carry

        lax.fori_loop(0, ANCH // 4, body, 0)

        for i in range(ACH):
            pltpu.make_async_copy(gbufs.at[1, i], table.at[loc_v.at[3, i]],
                                  ssems[1]).wait()
        for i in range(ACH):
            pltpu.make_async_copy(hs_hbm.at[src_v.at[0, i]],
                                  gbufs.at[0, i], gsems[0]).wait()
        idx_wait(ANCH - 1, 1, isems[1])
        idx_wait(ANCH - 1, 2, isems[0])

        plsc.subcore_barrier()
        for k in range(SEGS):
            pltpu.sync_copy(table.at[pl.ds(tb + k * 128, 128)],
                            out_hbm.at[c, pl.ds(tb + k * 128, 128)])

    return agg_kernel


_deg_call = _make_deg_kernel()
_agg32_call = _make_agg_kernel(HID)


BLK = 2000
RB = ROWS // 16


def _loc_body(dst_ref, src_ref, loc_ref, srcz_ref):
    i = pl.program_id(0)
    l = dst_ref[...] - i * HALF
    ok = (l >= 0) & (l < HALF)
    r = lax.broadcasted_iota(jnp.int32, l.shape, 0)
    q = lax.broadcasted_iota(jnp.int32, l.shape, 1)
    spread = (r * LANES + q) & 1023
    loc_ref[...] = jnp.where(ok, l, DUMMY + spread)[None]
    srcz_ref[...] = jnp.where(ok, src_ref[...], N_NODES + (spread & 127))[None]


def _loc_call(dstp, srcp):
    return pl.pallas_call(
        _loc_body,
        grid=(2, 16),
        in_specs=[
            pl.BlockSpec((RB, LANES), lambda i, j: (j, 0)),
            pl.BlockSpec((RB, LANES), lambda i, j: (j, 0)),
        ],
        out_specs=[
            pl.BlockSpec((1, RB, LANES), lambda i, j: (i, j, 0)),
            pl.BlockSpec((1, RB, LANES), lambda i, j: (i, j, 0)),
        ],
        out_shape=[
            jax.ShapeDtypeStruct((2, ROWS, LANES), jnp.int32),
            jax.ShapeDtypeStruct((2, ROWS, LANES), jnp.int32),
        ],
    )(dstp, srcp)


def _pre_body(x_ref, w_ref, deg_ref, hs_ref, s_ref):
    s = lax.rsqrt(deg_ref[0] + 1.0)
    h = jnp.dot(x_ref[...], w_ref[...], preferred_element_type=jnp.float32)
    hs_ref[...] = h * s
    s_ref[...] = s


def _pre_call(x, W1, deg3):
    return pl.pallas_call(
        _pre_body,
        grid=(2, HALF // BLK),
        in_specs=[
            pl.BlockSpec((BLK, F_IN), lambda i, j: (i * (HALF // BLK) + j, 0)),
            pl.BlockSpec((F_IN, HID), lambda i, j: (0, 0)),
            pl.BlockSpec((1, BLK, 1), lambda i, j: (i, j, 0)),
        ],
        out_specs=[
            pl.BlockSpec((BLK, HID), lambda i, j: (i * (HALF // BLK) + j, 0)),
            pl.BlockSpec((BLK, 1), lambda i, j: (i * (HALF // BLK) + j, 0)),
        ],
        out_shape=[
            jax.ShapeDtypeStruct((N_NODES, HID), jnp.float32),
            jax.ShapeDtypeStruct((N_NODES, 1), jnp.float32),
        ],
    )(x, W1, deg3)


def _mid_body(agg_ref, hs_ref, s_ref, b_ref, out_ref):
    s = s_ref[...]
    z = jnp.maximum(s * (agg_ref[0] + hs_ref[...]) + b_ref[...], 0.0)
    out_ref[...] = z * s


def _mid_call(agg1, hs1, sinv, b1r):
    return pl.pallas_call(
        _mid_body,
        grid=(2, HALF // BLK),
        in_specs=[
            pl.BlockSpec((1, BLK, HID), lambda i, j: (i, j, 0)),
            pl.BlockSpec((BLK, HID), lambda i, j: (i * (HALF // BLK) + j, 0)),
            pl.BlockSpec((BLK, 1), lambda i, j: (i * (HALF // BLK) + j, 0)),
            pl.BlockSpec((1, HID), lambda i, j: (0, 0)),
        ],
        out_specs=pl.BlockSpec((BLK, HID),
                               lambda i, j: (i * (HALF // BLK) + j, 0)),
        out_shape=jax.ShapeDtypeStruct((N_NODES, HID), jnp.float32),
    )(agg1, hs1, sinv, b1r)


def _post_body(agg_ref, zs_ref, s_ref, w_ref, b_ref, out_ref):
    s = s_ref[...]
    u = s * (agg_ref[0] + zs_ref[...])
    h2 = jnp.dot(u, w_ref[...], preferred_element_type=jnp.float32)
    z = jnp.maximum(h2 + b_ref[...], 0.0)
    m = jnp.max(z, axis=1, keepdims=True)
    lse = jnp.log(jnp.sum(jnp.exp(z - m), axis=1, keepdims=True)) + m
    out_ref[...] = z - lse


def _post_call(agg2, zs, sinv, W2, b2r):
    return pl.pallas_call(
        _post_body,
        grid=(2, HALF // BLK),
        in_specs=[
            pl.BlockSpec((1, BLK, HID), lambda i, j: (i, j, 0)),
            pl.BlockSpec((BLK, HID), lambda i, j: (i * (HALF // BLK) + j, 0)),
            pl.BlockSpec((BLK, 1), lambda i, j: (i * (HALF // BLK) + j, 0)),
            pl.BlockSpec((HID, NCLS), lambda i, j: (0, 0)),
            pl.BlockSpec((1, NCLS), lambda i, j: (0, 0)),
        ],
        out_specs=pl.BlockSpec((BLK, NCLS),
                               lambda i, j: (i * (HALF // BLK) + j, 0)),
        out_shape=jax.ShapeDtypeStruct((N_NODES, NCLS), jnp.float32),
    )(agg2, zs, sinv, W2, b2r)


def kernel(x, edge_index, W1, b1, W2, b2):
    pad = ROWS * LANES - N_EDGES
    srcp = jnp.concatenate(
        [edge_index[0], jnp.zeros((pad,), jnp.int32)]).reshape(ROWS, LANES)
    dstp = jnp.concatenate(
        [edge_index[1], jnp.full((pad,), N_NODES, jnp.int32)]).reshape(ROWS, LANES)

    locp, srcz = _loc_call(dstp, srcp)
    deg2 = _deg_call(locp).reshape(2, TR)
    deg3 = deg2[:, :HALF].reshape(2, HALF, 1)

    hs1, sinv = _pre_call(x, W1, deg3)
    zeros32 = jnp.zeros((LANES, HID), jnp.float32)
    hs1p = jnp.pad(hs1, ((0, PADZ), (0, 0)))
    agg1 = _agg32_call(hs1p, srcz, locp, zeros32)

    zs = _mid_call(agg1, hs1, sinv, b1.reshape(1, HID))
    zsp = jnp.pad(zs, ((0, PADZ), (0, 0)))
    agg2 = _agg32_call(zsp, srcz, locp, zeros32)

    return _post_call(agg2, zs, sinv, W2, b2.reshape(1, NCLS))
